# baseline (speedup 1.0000x reference)
import functools

import jax
import jax.numpy as jnp
from jax import lax
from jax.experimental import pallas as pl
from jax.experimental.pallas import tpu as pltpu
from jax.experimental.pallas import tpu_sc as plsc

N_IN = 10000
N_OUT = 10000
D = 128
L = 16
NC = 2
NS = 16
NW = NC * NS
SEG_PER_W = 320
LAST_SEG = N_OUT - SEG_PER_W * (NW - 1)
C = 128

_i32 = jnp.int32

_GATHER_DNUMS = lax.GatherDimensionNumbers(
    offset_dims=(), collapsed_slice_dims=(0,), start_index_map=(0,))


def _lane_splat(vec, lane_idx):
    return lax.gather(vec, lane_idx[:, None], _GATHER_DNUMS, (1,),
                      mode=lax.GatherScatterMode.PROMISE_IN_BOUNDS)

_mesh = plsc.VectorSubcoreMesh(core_axis_name="c", subcore_axis_name="s")


@functools.partial(
    pl.kernel,
    out_type=jax.ShapeDtypeStruct((N_OUT * D,), jnp.float32),
    mesh=_mesh,
    compiler_params=pltpu.CompilerParams(needs_layout_passes=False),
    scratch_types=[
        pltpu.VMEM((C,), jnp.int32),
        pltpu.VMEM((C, D), jnp.float32),
        pltpu.VMEM(((SEG_PER_W + 1) * D,), jnp.float32),
        pltpu.VMEM((C + L,), jnp.int32),
        pltpu.VMEM((48,), jnp.int32),
        pltpu.SemaphoreType.DMA,
    ],
)
def _sc_pool(feat_hbm, map_hbm, seg_hbm, bounds_hbm, out_hbm,
             idx_v, rows_v, out_local, seg_s, bounds_s, sem):
    cid = lax.axis_index("c")
    sid = lax.axis_index("s")
    wid = sid * _i32(NC) + cid

    pltpu.sync_copy(bounds_hbm, bounds_s)
    bvec = bounds_s[pl.ds(wid, L)]
    start = bvec[0]
    end = bvec[1]
    lo = wid * _i32(SEG_PER_W)

    zeros16 = jnp.zeros((L,), jnp.float32)

    def zero_body(i, carry):
        out_local[pl.ds(i * _i32(L), L)] = zeros16
        return carry

    lax.fori_loop(_i32(0), _i32(SEG_PER_W * D // L), zero_body, _i32(0))

    base = (start // _i32(8)) * _i32(8)
    nchunks = (end - base + _i32(C - 1)) // _i32(C)

    def chunk_body(k, carry):
        p = pl.multiple_of(base + k * _i32(C), 8)
        pltpu.sync_copy(map_hbm.at[pl.ds(p, C)], idx_v)
        pltpu.sync_copy(seg_hbm.at[pl.ds(p, C)], seg_s.at[pl.ds(0, C)])
        pltpu.async_copy(feat_hbm.at[idx_v], rows_v, sem).wait()

        def group_body(g, gc):
            cur = gc[0]
            acc = list(gc[1:])
            jb = g * _i32(L)
            sv = seg_s[pl.ds(jb, L)]
            for l in range(L):
                s = sv[l]
                same = s == cur
                wbase = jnp.minimum(((s - lo) * _i32(D)).astype(jnp.uint32),
                                    jnp.uint32(SEG_PER_W * D)).astype(jnp.int32)
                jrow = jb + _i32(l)
                for c in range(D // L):
                    row = rows_v[jrow, pl.ds(c * L, L)]
                    a = lax.select(same, jnp.maximum(acc[c], row), row)
                    acc[c] = a
                    out_local[pl.ds(wbase + _i32(c * L), L)] = a
                cur = s
            return (cur, *acc)

        return lax.fori_loop(_i32(0), _i32(C // L), group_body, carry)

    init = (jnp.int32(-1),) + tuple(
        jnp.full((L,), -jnp.inf, jnp.float32) for _ in range(D // L))
    lax.fori_loop(_i32(0), nchunks, chunk_body, init)

    @pl.when(wid < _i32(NW - 1))
    def _write_full():
        pltpu.sync_copy(out_local.at[pl.ds(0, SEG_PER_W * D)],
                        out_hbm.at[pl.ds(lo * _i32(D), SEG_PER_W * D)])

    @pl.when(wid == _i32(NW - 1))
    def _write_last():
        pltpu.sync_copy(out_local.at[pl.ds(0, LAST_SEG * D)],
                        out_hbm.at[pl.ds(lo * _i32(D), LAST_SEG * D)])


def kernel(in_feat, in_map, seg_ids):
    map32 = in_map.astype(jnp.int32)
    seg32 = seg_ids.astype(jnp.int32)
    targets = jnp.arange(NW + 1, dtype=jnp.int32) * SEG_PER_W
    bounds = jnp.searchsorted(seg32, targets, side="left").astype(jnp.int32)
    bounds = jnp.pad(bounds, (0, 48 - (NW + 1)))
    map_p = jnp.concatenate([map32, jnp.zeros((C,), jnp.int32)])
    seg_p = jnp.concatenate([seg32, jnp.full((C,), N_OUT, jnp.int32)])
    out = _sc_pool(in_feat.astype(jnp.float32), map_p, seg_p, bounds)
    return out.reshape(N_OUT, D)


# speedup vs baseline: 1.8589x; 1.8589x over previous
import functools

import jax
import jax.numpy as jnp
from jax import lax
from jax.experimental import pallas as pl
from jax.experimental.pallas import tpu as pltpu
from jax.experimental.pallas import tpu_sc as plsc

N_IN = 10000
N_OUT = 10000
D = 128
L = 16
NC = 2
NS = 16
NW = NC * NS
SEG_PER_W = 320
LAST_SEG = N_OUT - SEG_PER_W * (NW - 1)
C = 128

_i32 = jnp.int32

_GATHER_DNUMS = lax.GatherDimensionNumbers(
    offset_dims=(), collapsed_slice_dims=(0,), start_index_map=(0,))


def _lane_splat(vec, lane_idx):
    return lax.gather(vec, lane_idx[:, None], _GATHER_DNUMS, (1,),
                      mode=lax.GatherScatterMode.PROMISE_IN_BOUNDS)

_mesh = plsc.VectorSubcoreMesh(core_axis_name="c", subcore_axis_name="s")


@functools.partial(
    pl.kernel,
    out_type=jax.ShapeDtypeStruct((N_OUT * D,), jnp.float32),
    mesh=_mesh,
    compiler_params=pltpu.CompilerParams(needs_layout_passes=False),
    scratch_types=[
        pltpu.VMEM((C,), jnp.int32),
        pltpu.VMEM((C, D), jnp.float32),
        pltpu.VMEM(((SEG_PER_W + 1) * D,), jnp.float32),
        pltpu.VMEM((C + L,), jnp.int32),
        pltpu.VMEM((48,), jnp.int32),
        pltpu.SemaphoreType.DMA,
    ],
)
def _sc_pool(feat_hbm, map_hbm, seg_hbm, bounds_hbm, out_hbm,
             idx_v, rows_v, out_local, seg_s, bounds_s, sem):
    cid = lax.axis_index("c")
    sid = lax.axis_index("s")
    wid = sid * _i32(NC) + cid

    pltpu.sync_copy(bounds_hbm, bounds_s)
    bvec = bounds_s[pl.ds(wid, L)]
    start = bvec[0]
    end = bvec[1]
    lo = wid * _i32(SEG_PER_W)

    zeros16 = jnp.zeros((L,), jnp.float32)

    def zero_body(i, carry):
        out_local[pl.ds(i * _i32(L), L)] = zeros16
        return carry

    lax.fori_loop(_i32(0), _i32(SEG_PER_W * D // L), zero_body, _i32(0))

    base = (start // _i32(8)) * _i32(8)
    nchunks = (end - base + _i32(C - 1)) // _i32(C)

    def clamp_base(seg_val):
        return jnp.minimum(((seg_val - lo) * _i32(D)).astype(jnp.uint32),
                           jnp.uint32(SEG_PER_W * D)).astype(jnp.int32)

    def chunk_body(k, carry):
        p = pl.multiple_of(base + k * _i32(C), 8)
        pltpu.sync_copy(map_hbm.at[pl.ds(p, C)], idx_v)
        pltpu.sync_copy(seg_hbm.at[pl.ds(p, C)], seg_s.at[pl.ds(0, C)])
        pltpu.async_copy(feat_hbm.at[idx_v], rows_v, sem).wait()

        def group_body(g, gc):
            cur = gc[0]
            acc = list(gc[1:])
            jb = g * _i32(L)
            sv = seg_s[pl.ds(jb, L)]
            for l in range(L):
                s = sv[l]
                changed = s != cur
                wbase = clamp_base(cur)

                @pl.when(changed)
                def _flush(wbase=wbase, snap=tuple(acc)):
                    for c in range(D // L):
                        out_local[pl.ds(wbase + _i32(c * L), L)] = snap[c]

                jrow = jb + _i32(l)
                for c in range(D // L):
                    row = rows_v[jrow, pl.ds(c * L, L)]
                    acc[c] = lax.select(changed, row, jnp.maximum(acc[c], row))
                cur = s
            return (cur, *acc)

        return lax.fori_loop(_i32(0), _i32(C // L), group_body, carry)

    init = (jnp.int32(-1),) + tuple(
        jnp.full((L,), -jnp.inf, jnp.float32) for _ in range(D // L))
    final = lax.fori_loop(_i32(0), nchunks, chunk_body, init)
    cur = final[0]
    acc = final[1:]
    wb_final = clamp_base(cur)
    for c in range(D // L):
        out_local[pl.ds(wb_final + _i32(c * L), L)] = acc[c]

    @pl.when(wid < _i32(NW - 1))
    def _write_full():
        pltpu.sync_copy(out_local.at[pl.ds(0, SEG_PER_W * D)],
                        out_hbm.at[pl.ds(lo * _i32(D), SEG_PER_W * D)])

    @pl.when(wid == _i32(NW - 1))
    def _write_last():
        pltpu.sync_copy(out_local.at[pl.ds(0, LAST_SEG * D)],
                        out_hbm.at[pl.ds(lo * _i32(D), LAST_SEG * D)])


def kernel(in_feat, in_map, seg_ids):
    map32 = in_map.astype(jnp.int32)
    seg32 = seg_ids.astype(jnp.int32)
    targets = jnp.arange(NW + 1, dtype=jnp.int32) * SEG_PER_W
    bounds = jnp.searchsorted(seg32, targets, side="left").astype(jnp.int32)
    bounds = jnp.pad(bounds, (0, 48 - (NW + 1)))
    map_p = jnp.concatenate([map32, jnp.zeros((C,), jnp.int32)])
    seg_p = jnp.concatenate([seg32, jnp.full((C,), N_OUT, jnp.int32)])
    out = _sc_pool(in_feat.astype(jnp.float32), map_p, seg_p, bounds)
    return out.reshape(N_OUT, D)


# speedup vs baseline: 3.8805x; 2.0876x over previous
import functools

import jax
import jax.numpy as jnp
from jax import lax
from jax.experimental import pallas as pl
from jax.experimental.pallas import tpu as pltpu
from jax.experimental.pallas import tpu_sc as plsc

N_IN = 10000
N_OUT = 10000
D = 128
L = 16
NC = 2
NS = 16
NW = NC * NS
SEG_PER_W = 320
LAST_SEG = N_OUT - SEG_PER_W * (NW - 1)
C = 128

_i32 = jnp.int32

_GATHER_DNUMS = lax.GatherDimensionNumbers(
    offset_dims=(), collapsed_slice_dims=(0,), start_index_map=(0,))


def _lane_splat(vec, lane_idx):
    return lax.gather(vec, lane_idx[:, None], _GATHER_DNUMS, (1,),
                      mode=lax.GatherScatterMode.PROMISE_IN_BOUNDS)

_mesh = plsc.VectorSubcoreMesh(core_axis_name="c", subcore_axis_name="s")


@functools.partial(
    pl.kernel,
    out_type=jax.ShapeDtypeStruct((N_OUT * D,), jnp.float32),
    mesh=_mesh,
    compiler_params=pltpu.CompilerParams(needs_layout_passes=False),
    scratch_types=[
        pltpu.VMEM((2, C), jnp.int32),
        pltpu.VMEM((2, C, D), jnp.float32),
        pltpu.VMEM(((SEG_PER_W + 1) * D,), jnp.float32),
        pltpu.VMEM((2, C), jnp.int32),
        pltpu.VMEM((48,), jnp.int32),
        pltpu.SemaphoreType.DMA((2,)),
        pltpu.SemaphoreType.DMA((2,)),
    ],
)
def _sc_pool(feat_hbm, map_hbm, seg_hbm, bounds_hbm, out_hbm,
             idx_v, rows_v, out_local, seg_s, bounds_s, sem_is, sem_g):
    cid = lax.axis_index("c")
    sid = lax.axis_index("s")
    wid = sid * _i32(NC) + cid

    pltpu.sync_copy(bounds_hbm, bounds_s)
    bvec = bounds_s[pl.ds(wid, L)]
    start = bvec[0]
    end = bvec[1]
    lo = wid * _i32(SEG_PER_W)

    zeros16 = jnp.zeros((L,), jnp.float32)

    def zero_body(i, carry):
        out_local[pl.ds(i * _i32(L), L)] = zeros16
        return carry

    lax.fori_loop(_i32(0), _i32(SEG_PER_W * D // L), zero_body, _i32(0))

    base = (start // _i32(8)) * _i32(8)
    nchunks = (end - base + _i32(C - 1)) // _i32(C)

    def clamp_base(seg_val):
        return jnp.minimum(((seg_val - lo) * _i32(D)).astype(jnp.uint32),
                           jnp.uint32(SEG_PER_W * D)).astype(jnp.int32)

    n = nchunks

    def start_is(m, s):
        pm = pl.multiple_of(base + m * _i32(C), 8)
        pltpu.async_copy(map_hbm.at[pl.ds(pm, C)], idx_v.at[s], sem_is.at[s])
        pltpu.async_copy(seg_hbm.at[pl.ds(pm, C)], seg_s.at[s], sem_is.at[s])

    def wait_is(s):
        pltpu.make_async_copy(map_hbm.at[pl.ds(0, C)], idx_v.at[s],
                              sem_is.at[s]).wait()
        pltpu.make_async_copy(seg_hbm.at[pl.ds(0, C)], seg_s.at[s],
                              sem_is.at[s]).wait()

    def start_g(s):
        pltpu.async_copy(feat_hbm.at[idx_v.at[s]], rows_v.at[s], sem_g.at[s])

    def wait_g(s):
        pltpu.make_async_copy(feat_hbm.at[idx_v.at[s]], rows_v.at[s],
                              sem_g.at[s]).wait()

    @pl.when(n > _i32(0))
    def _prologue():
        start_is(_i32(0), _i32(0))
        wait_is(_i32(0))
        start_g(_i32(0))

        @pl.when(n > _i32(1))
        def _():
            start_is(_i32(1), _i32(1))

    def chunk_body(k, carry):
        s = jnp.bitwise_and(k, _i32(1))
        s1 = _i32(1) - s

        @pl.when(k + _i32(1) < n)
        def _():
            wait_is(s1)
            start_g(s1)

        wait_g(s)

        @pl.when(k + _i32(2) < n)
        def _():
            start_is(k + _i32(2), s)

        def group_body(g, gc):
            cur = gc[0]
            acc = list(gc[1:])
            jb = g * _i32(L)
            sv = seg_s[s, pl.ds(jb, L)]
            for l in range(L):
                sval = sv[l]
                changed = sval != cur
                wbase = clamp_base(cur)

                @pl.when(changed)
                def _flush(wbase=wbase, snap=tuple(acc)):
                    for c in range(D // L):
                        out_local[pl.ds(wbase + _i32(c * L), L)] = snap[c]

                jrow = jb + _i32(l)
                for c in range(D // L):
                    row = rows_v[s, jrow, pl.ds(c * L, L)]
                    acc[c] = lax.select(changed, row, jnp.maximum(acc[c], row))
                cur = sval
            return (cur, *acc)

        return lax.fori_loop(_i32(0), _i32(C // L), group_body, carry)

    init = (jnp.int32(-1),) + tuple(
        jnp.full((L,), -jnp.inf, jnp.float32) for _ in range(D // L))
    final = lax.fori_loop(_i32(0), nchunks, chunk_body, init)
    cur = final[0]
    acc = final[1:]
    wb_final = clamp_base(cur)
    for c in range(D // L):
        out_local[pl.ds(wb_final + _i32(c * L), L)] = acc[c]

    @pl.when(wid < _i32(NW - 1))
    def _write_full():
        pltpu.sync_copy(out_local.at[pl.ds(0, SEG_PER_W * D)],
                        out_hbm.at[pl.ds(lo * _i32(D), SEG_PER_W * D)])

    @pl.when(wid == _i32(NW - 1))
    def _write_last():
        pltpu.sync_copy(out_local.at[pl.ds(0, LAST_SEG * D)],
                        out_hbm.at[pl.ds(lo * _i32(D), LAST_SEG * D)])


def kernel(in_feat, in_map, seg_ids):
    map32 = in_map.astype(jnp.int32)
    seg32 = seg_ids.astype(jnp.int32)
    targets = jnp.arange(NW + 1, dtype=jnp.int32) * SEG_PER_W
    bounds = jnp.searchsorted(seg32, targets, side="left").astype(jnp.int32)
    bounds = jnp.pad(bounds, (0, 48 - (NW + 1)))
    map_p = jnp.concatenate([map32, jnp.zeros((C,), jnp.int32)])
    seg_p = jnp.concatenate([seg32, jnp.full((C,), N_OUT, jnp.int32)])
    out = _sc_pool(in_feat.astype(jnp.float32), map_p, seg_p, bounds)
    return out.reshape(N_OUT, D)
